# Initial kernel scaffold; baseline (speedup 1.0000x reference)
import functools

import jax
import jax.numpy as jnp
from jax import lax
from jax.experimental import pallas as pl
from jax.experimental.pallas import tpu as pltpu
from jax.experimental.pallas import tpu_sc as plsc

_GAMMA = 12.0
_E = 320000
_D = 128
_L = 16

_info = plsc.get_sparse_core_info()
_NC = _info.num_cores
_NS = _info.num_subcores
_NW = _NC * _NS
_E_PER_W = _E // _NW
_BLK = 80
_NBLK = _E_PER_W // _BLK


def _make_sc_kernel():
    mesh = plsc.VectorSubcoreMesh(core_axis_name="c", subcore_axis_name="s")

    @functools.partial(
        pl.kernel,
        mesh=mesh,
        out_type=jax.ShapeDtypeStruct((_E,), jnp.float32),
        scratch_types=[
            pltpu.VMEM((_E_PER_W,), jnp.int32),
            pltpu.VMEM((_E_PER_W,), jnp.int32),
            pltpu.VMEM((_E_PER_W,), jnp.float32),
            pltpu.VMEM((_BLK, _D), jnp.float32),
            pltpu.VMEM((_BLK, _D), jnp.float32),
            pltpu.VMEM((_BLK, _D), jnp.float32),
            pltpu.SemaphoreType.DMA,
            pltpu.SemaphoreType.DMA,
        ],
    )
    def sc_kernel(x_hbm, src_hbm, dst_hbm, rel_hbm, out_hbm,
                  sidx_v, didx_v, out_v, head_v, tail_v, rel_v,
                  sem_g, sem_l):
        wid = lax.axis_index("s") * _NC + lax.axis_index("c")
        base = wid * _E_PER_W
        pltpu.sync_copy(src_hbm.at[pl.ds(base, _E_PER_W)], sidx_v)
        pltpu.sync_copy(dst_hbm.at[pl.ds(base, _E_PER_W)], didx_v)

        lane = lax.iota(jnp.int32, _L)

        def block_body(b, carry):
            off = b * _BLK
            cg1 = pltpu.async_copy(
                x_hbm.at[sidx_v.at[pl.ds(off, _BLK)]], head_v, sem_g)
            cg2 = pltpu.async_copy(
                x_hbm.at[didx_v.at[pl.ds(off, _BLK)]], tail_v, sem_g)
            cl = pltpu.async_copy(
                rel_hbm.at[pl.ds(base + off, _BLK)], rel_v, sem_l)
            cg1.wait()
            cg2.wait()
            cl.wait()

            def group_body(g, carry2):
                row0 = g * _L
                vec = jnp.zeros((_L,), jnp.float32)
                for e in range(_L):
                    r = row0 + e
                    acc = jnp.zeros((_L,), jnp.float32)
                    for j in range(_D // _L):
                        h = head_v[r, pl.ds(j * _L, _L)]
                        re = rel_v[r, pl.ds(j * _L, _L)]
                        t = tail_v[r, pl.ds(j * _L, _L)]
                        acc = acc + jnp.abs(h + re - t)
                    s = jnp.sum(acc)
                    vec = jnp.where(lane == e, _GAMMA - s, vec)
                out_v[pl.ds(off + row0, _L)] = vec
                return carry2

            lax.fori_loop(0, _BLK // _L, group_body, 0)
            return carry

        lax.fori_loop(0, _NBLK, block_body, 0)
        pltpu.sync_copy(out_v, out_hbm.at[pl.ds(base, _E_PER_W)])

    return sc_kernel


_sc_kernel = _make_sc_kernel()


@jax.jit
def kernel(x, edge_index, rel_emb):
    src = edge_index[0].astype(jnp.int32)
    dst = edge_index[1].astype(jnp.int32)
    return _sc_kernel(x, src, dst, rel_emb)


# speedup vs baseline: 2.4347x; 2.4347x over previous
import functools

import jax
import jax.numpy as jnp
from jax import lax
from jax.experimental import pallas as pl
from jax.experimental.pallas import tpu as pltpu
from jax.experimental.pallas import tpu_sc as plsc

_GAMMA = 12.0
_E = 320000
_D = 128
_L = 16

_info = plsc.get_sparse_core_info()
_NC = _info.num_cores
_NS = _info.num_subcores
_NW = _NC * _NS
_E_PER_W = _E // _NW
_BLK = 80
_NBLK = _E_PER_W // _BLK


def _make_sc_kernel():
    mesh = plsc.VectorSubcoreMesh(core_axis_name="c", subcore_axis_name="s")

    @functools.partial(
        pl.kernel,
        mesh=mesh,
        out_type=jax.ShapeDtypeStruct((_E,), jnp.float32),
        scratch_types=[
            pltpu.VMEM((_E_PER_W,), jnp.int32),
            pltpu.VMEM((_E_PER_W,), jnp.int32),
            pltpu.VMEM((_E_PER_W,), jnp.float32),
            pltpu.VMEM((_BLK, _D), jnp.float32),
            pltpu.VMEM((_BLK, _D), jnp.float32),
            pltpu.VMEM((_BLK, _D), jnp.float32),
            pltpu.SemaphoreType.DMA,
            pltpu.SemaphoreType.DMA,
        ],
    )
    def sc_kernel(x_hbm, src_hbm, dst_hbm, rel_hbm, out_hbm,
                  sidx_v, didx_v, out_v, head_v, tail_v, rel_v,
                  sem_g, sem_l):
        wid = lax.axis_index("s") * _NC + lax.axis_index("c")
        base = wid * _E_PER_W
        pltpu.sync_copy(src_hbm.at[pl.ds(base, _E_PER_W)], sidx_v)
        pltpu.sync_copy(dst_hbm.at[pl.ds(base, _E_PER_W)], didx_v)

        lane = lax.iota(jnp.int32, _L)
        rot_idx = [(lane + sh) & (_L - 1) for sh in (8, 4, 2, 1)]
        gd = lax.GatherDimensionNumbers(
            offset_dims=(), collapsed_slice_dims=(0,), start_index_map=(0,))

        def _rot(v, idx):
            return lax.gather(v, idx[:, None], gd, (1,),
                              mode=lax.GatherScatterMode.PROMISE_IN_BOUNDS)

        def block_body(b, carry):
            off = b * _BLK
            cg1 = pltpu.async_copy(
                x_hbm.at[sidx_v.at[pl.ds(off, _BLK)]], head_v, sem_g)
            cg2 = pltpu.async_copy(
                x_hbm.at[didx_v.at[pl.ds(off, _BLK)]], tail_v, sem_g)
            cl = pltpu.async_copy(
                rel_hbm.at[pl.ds(base + off, _BLK)], rel_v, sem_l)
            cg1.wait()
            cg2.wait()
            cl.wait()

            def group_body(g, carry2):
                row0 = g * _L
                vec = jnp.zeros((_L,), jnp.float32)
                for e in range(_L):
                    r = row0 + e
                    acc = jnp.zeros((_L,), jnp.float32)
                    for j in range(_D // _L):
                        h = head_v[r, pl.ds(j * _L, _L)]
                        re = rel_v[r, pl.ds(j * _L, _L)]
                        t = tail_v[r, pl.ds(j * _L, _L)]
                        acc = acc + jnp.abs(h + re - t)
                    for idx in rot_idx:
                        acc = acc + _rot(acc, idx)
                    vec = jnp.where(lane == e, _GAMMA - acc, vec)
                out_v[pl.ds(off + row0, _L)] = vec
                return carry2

            lax.fori_loop(0, _BLK // _L, group_body, 0)
            return carry

        lax.fori_loop(0, _NBLK, block_body, 0)
        pltpu.sync_copy(out_v, out_hbm.at[pl.ds(base, _E_PER_W)])

    return sc_kernel


_sc_kernel = _make_sc_kernel()


@jax.jit
def kernel(x, edge_index, rel_emb):
    src = edge_index[0].astype(jnp.int32)
    dst = edge_index[1].astype(jnp.int32)
    return _sc_kernel(x, src, dst, rel_emb)


# speedup vs baseline: 4.5405x; 1.8649x over previous
import functools

import jax
import jax.numpy as jnp
from jax import lax
from jax.experimental import pallas as pl
from jax.experimental.pallas import tpu as pltpu
from jax.experimental.pallas import tpu_sc as plsc

_GAMMA = 12.0
_E = 320000
_D = 128
_L = 16

_info = plsc.get_sparse_core_info()
_NC = _info.num_cores
_NS = _info.num_subcores
_NW = _NC * _NS
_E_PER_W = _E // _NW
_BLK = 80
_NBLK = _E_PER_W // _BLK


def _make_sc_kernel():
    mesh = plsc.VectorSubcoreMesh(core_axis_name="c", subcore_axis_name="s")

    @functools.partial(
        pl.kernel,
        mesh=mesh,
        out_type=jax.ShapeDtypeStruct((_E,), jnp.float32),
        scratch_types=[
            pltpu.VMEM((_E_PER_W,), jnp.int32),
            pltpu.VMEM((_E_PER_W,), jnp.int32),
            pltpu.VMEM((_E_PER_W,), jnp.float32),
            pltpu.VMEM((2, _BLK, _D), jnp.float32),
            pltpu.VMEM((2, _BLK, _D), jnp.float32),
            pltpu.VMEM((2, _BLK, _D), jnp.float32),
            pltpu.SemaphoreType.DMA,
            pltpu.SemaphoreType.DMA,
        ],
    )
    def sc_kernel(x_hbm, src_hbm, dst_hbm, rel_hbm, out_hbm,
                  sidx_v, didx_v, out_v, head_v, tail_v, rel_v, sem0, sem1):
        wid = lax.axis_index("s") * _NC + lax.axis_index("c")
        base = wid * _E_PER_W
        pltpu.sync_copy(src_hbm.at[pl.ds(base, _E_PER_W)], sidx_v)
        pltpu.sync_copy(dst_hbm.at[pl.ds(base, _E_PER_W)], didx_v)

        sems = (sem0, sem1)
        lane = lax.iota(jnp.int32, _L)
        rot_idx = [(lane + sh) & (_L - 1) for sh in (8, 4, 2, 1)]
        gd = lax.GatherDimensionNumbers(
            offset_dims=(), collapsed_slice_dims=(0,), start_index_map=(0,))

        def _rot(v, idx):
            return lax.gather(v, idx[:, None], gd, (1,),
                              mode=lax.GatherScatterMode.PROMISE_IN_BOUNDS)

        def start(b, slot):
            off = b * _BLK
            pltpu.async_copy(
                x_hbm.at[sidx_v.at[pl.ds(off, _BLK)]],
                head_v.at[slot], sems[slot])
            pltpu.async_copy(
                x_hbm.at[didx_v.at[pl.ds(off, _BLK)]],
                tail_v.at[slot], sems[slot])
            pltpu.async_copy(
                rel_hbm.at[pl.ds(base + off, _BLK)],
                rel_v.at[slot], sems[slot])

        def wait(b, slot):
            off = b * _BLK
            pltpu.make_async_copy(
                x_hbm.at[sidx_v.at[pl.ds(off, _BLK)]],
                head_v.at[slot], sems[slot]).wait()
            pltpu.make_async_copy(
                x_hbm.at[didx_v.at[pl.ds(off, _BLK)]],
                tail_v.at[slot], sems[slot]).wait()
            pltpu.make_async_copy(
                rel_hbm.at[pl.ds(base + off, _BLK)],
                rel_v.at[slot], sems[slot]).wait()

        def compute(b, slot):
            off = b * _BLK
            hb, tb, rb = head_v.at[slot], tail_v.at[slot], rel_v.at[slot]

            def group_body(g, carry2):
                row0 = g * _L
                vec = jnp.zeros((_L,), jnp.float32)
                for e in range(_L):
                    r = row0 + e
                    acc = jnp.zeros((_L,), jnp.float32)
                    for j in range(_D // _L):
                        h = hb[r, pl.ds(j * _L, _L)]
                        re = rb[r, pl.ds(j * _L, _L)]
                        t = tb[r, pl.ds(j * _L, _L)]
                        acc = acc + jnp.abs(h + re - t)
                    for idx in rot_idx:
                        acc = acc + _rot(acc, idx)
                    vec = jnp.where(lane == e, _GAMMA - acc, vec)
                out_v[pl.ds(off + row0, _L)] = vec
                return carry2

            lax.fori_loop(0, _BLK // _L, group_body, 0)

        start(0, 0)

        def pair_body(g, carry):
            b0 = 2 * g
            start(b0 + 1, 1)
            wait(b0, 0)
            compute(b0, 0)
            start(b0 + 2, 0)
            wait(b0 + 1, 1)
            compute(b0 + 1, 1)
            return carry

        lax.fori_loop(0, (_NBLK - 1) // 2, pair_body, 0)
        wait(_NBLK - 1, 0)
        compute(_NBLK - 1, 0)

        pltpu.sync_copy(out_v, out_hbm.at[pl.ds(base, _E_PER_W)])

    return sc_kernel


_sc_kernel = _make_sc_kernel()


@jax.jit
def kernel(x, edge_index, rel_emb):
    src = edge_index[0].astype(jnp.int32)
    dst = edge_index[1].astype(jnp.int32)
    return _sc_kernel(x, src, dst, rel_emb)


# speedup vs baseline: 7.4035x; 1.6305x over previous
import functools

import jax
import jax.numpy as jnp
from jax import lax
from jax.experimental import pallas as pl
from jax.experimental.pallas import tpu as pltpu
from jax.experimental.pallas import tpu_sc as plsc

_GAMMA = 12.0
_E = 320000
_D = 128
_L = 16
_N_PAD = 10240

_info = plsc.get_sparse_core_info()
_NC = _info.num_cores
_NS = _info.num_subcores
_NW = _NC * _NS
_E_PER_W = _E // _NW
_BLK = 80
_NBLK = _E_PER_W // _BLK


def _make_sc_kernel():
    mesh = plsc.VectorSubcoreMesh(core_axis_name="c", subcore_axis_name="s")

    @functools.partial(
        pl.kernel,
        mesh=mesh,
        out_type=jax.ShapeDtypeStruct((_E,), jnp.float32),
        compiler_params=pltpu.CompilerParams(use_tc_tiling_on_sc=False),
        scratch_types=[
            pltpu.VMEM((_E_PER_W,), jnp.int32),
            pltpu.VMEM((_E_PER_W,), jnp.int32),
            pltpu.VMEM((2, _BLK), jnp.float32),
            pltpu.VMEM((2, _BLK, _D // 2), jnp.int32),
            pltpu.VMEM((2, _BLK, _D // 2), jnp.int32),
            pltpu.VMEM((2, _BLK, _D), jnp.float32),
            pltpu.SemaphoreType.DMA,
            pltpu.SemaphoreType.DMA,
            pltpu.SemaphoreType.DMA,
            pltpu.SemaphoreType.DMA,
        ],
    )
    def sc_kernel(x_hbm, src_hbm, dst_hbm, rel_hbm, out_hbm,
                  sidx_v, didx_v, out_ring, head_v, tail_v, rel_v,
                  sem0, sem1, sem_o0, sem_o1):
        wid = lax.axis_index("s") * _NC + lax.axis_index("c")
        base = wid * _E_PER_W

        pltpu.sync_copy(src_hbm.at[pl.ds(base, _E_PER_W)], sidx_v)
        pltpu.sync_copy(dst_hbm.at[pl.ds(base, _E_PER_W)], didx_v)

        sems = (sem0, sem1)
        out_sems = (sem_o0, sem_o1)
        lane = lax.iota(jnp.int32, _L)
        rot_idx = [(lane + sh) & (_L - 1) for sh in (8, 4, 2, 1)]
        gd = lax.GatherDimensionNumbers(
            offset_dims=(), collapsed_slice_dims=(0,), start_index_map=(0,))

        def _rot(v, idx):
            return lax.gather(v, idx[:, None], gd, (1,),
                              mode=lax.GatherScatterMode.PROMISE_IN_BOUNDS)

        def start(b, slot):
            off = b * _BLK
            pltpu.async_copy(
                x_hbm.at[sidx_v.at[pl.ds(off, _BLK)]],
                head_v.at[slot], sems[slot])
            pltpu.async_copy(
                x_hbm.at[didx_v.at[pl.ds(off, _BLK)]],
                tail_v.at[slot], sems[slot])
            pltpu.async_copy(
                rel_hbm.at[pl.ds(base + off, _BLK)],
                rel_v.at[slot], sems[slot])

        def wait(b, slot):
            off = b * _BLK
            pltpu.make_async_copy(
                x_hbm.at[sidx_v.at[pl.ds(off, _BLK)]],
                head_v.at[slot], sems[slot]).wait()
            pltpu.make_async_copy(
                x_hbm.at[didx_v.at[pl.ds(off, _BLK)]],
                tail_v.at[slot], sems[slot]).wait()
            pltpu.make_async_copy(
                rel_hbm.at[pl.ds(base + off, _BLK)],
                rel_v.at[slot], sems[slot]).wait()

        def compute(b, slot):
            off = b * _BLK
            hb, tb, rb = head_v.at[slot], tail_v.at[slot], rel_v.at[slot]
            ob = out_ring.at[slot]

            @pl.when(b >= 2)
            def _():
                pltpu.make_async_copy(
                    ob, out_hbm.at[pl.ds(base + (b - 2) * _BLK, _BLK)],
                    out_sems[slot]).wait()

            hi_mask = jnp.full((_L,), -65536, jnp.int32)

            def group_body(g, carry2):
                row0 = g * _L
                vec = jnp.zeros((_L,), jnp.float32)
                for e in range(_L):
                    r = row0 + e
                    acc = jnp.zeros((_L,), jnp.float32)
                    for c in range(_D // (2 * _L)):
                        hw = hb[r, pl.ds(c * _L, _L)]
                        tw = tb[r, pl.ds(c * _L, _L)]
                        h0 = lax.bitcast_convert_type(hw << 16, jnp.float32)
                        h1 = lax.bitcast_convert_type(hw & hi_mask, jnp.float32)
                        t0 = lax.bitcast_convert_type(tw << 16, jnp.float32)
                        t1 = lax.bitcast_convert_type(tw & hi_mask, jnp.float32)
                        r0 = rb[r, pl.ds((2 * c) * _L, _L)]
                        r1 = rb[r, pl.ds((2 * c + 1) * _L, _L)]
                        acc = acc + jnp.abs(h0 + r0 - t0)
                        acc = acc + jnp.abs(h1 + r1 - t1)
                    for idx in rot_idx:
                        acc = acc + _rot(acc, idx)
                    vec = jnp.where(lane == e, _GAMMA - acc, vec)
                ob[pl.ds(row0, _L)] = vec
                return carry2

            lax.fori_loop(0, _BLK // _L, group_body, 0)
            pltpu.async_copy(
                ob, out_hbm.at[pl.ds(base + off, _BLK)], out_sems[slot])

        start(0, 0)

        def pair_body(g, carry):
            b0 = 2 * g
            start(b0 + 1, 1)
            wait(b0, 0)
            compute(b0, 0)
            start(b0 + 2, 0)
            wait(b0 + 1, 1)
            compute(b0 + 1, 1)
            return carry

        lax.fori_loop(0, (_NBLK - 1) // 2, pair_body, 0)
        wait(_NBLK - 1, 0)
        compute(_NBLK - 1, 0)

        pltpu.make_async_copy(
            out_ring.at[1],
            out_hbm.at[pl.ds(base + (_NBLK - 2) * _BLK, _BLK)],
            out_sems[1]).wait()
        pltpu.make_async_copy(
            out_ring.at[0],
            out_hbm.at[pl.ds(base + (_NBLK - 1) * _BLK, _BLK)],
            out_sems[0]).wait()

    return sc_kernel


_sc_kernel = _make_sc_kernel()


@jax.jit
def kernel(x, edge_index, rel_emb):
    src = edge_index[0].astype(jnp.int32)
    dst = edge_index[1].astype(jnp.int32)
    n = x.shape[0]
    x_perm = x.reshape(n, _D // 32, 2, _L).swapaxes(2, 3).reshape(n, _D)
    x_bf = jnp.pad(x_perm, ((0, _N_PAD - n), (0, 0))).astype(jnp.bfloat16)
    x_i32 = jax.lax.bitcast_convert_type(
        x_bf.reshape(_N_PAD, _D // 2, 2), jnp.int32)
    return _sc_kernel(x_i32, src, dst, rel_emb)


# speedup vs baseline: 7.6113x; 1.0281x over previous
import functools

import jax
import jax.numpy as jnp
from jax import lax
from jax.experimental import pallas as pl
from jax.experimental.pallas import tpu as pltpu
from jax.experimental.pallas import tpu_sc as plsc

_GAMMA = 12.0
_E = 320000
_D = 128
_L = 16
_N_PAD = 10240

_info = plsc.get_sparse_core_info()
_NC = _info.num_cores
_NS = _info.num_subcores
_NW = _NC * _NS
_E_PER_W = _E // _NW
_BLK = 80
_NBLK = _E_PER_W // _BLK


def _make_sc_kernel():
    mesh = plsc.VectorSubcoreMesh(core_axis_name="c", subcore_axis_name="s")

    @functools.partial(
        pl.kernel,
        mesh=mesh,
        out_type=jax.ShapeDtypeStruct((_E,), jnp.float32),
        compiler_params=pltpu.CompilerParams(
            use_tc_tiling_on_sc=False, needs_layout_passes=False),
        scratch_types=[
            pltpu.VMEM((_E_PER_W,), jnp.int32),
            pltpu.VMEM((_E_PER_W,), jnp.int32),
            pltpu.VMEM((2, _BLK), jnp.float32),
            pltpu.VMEM((2, _BLK, _D // 2), jnp.int32),
            pltpu.VMEM((2, _BLK, _D // 2), jnp.int32),
            pltpu.VMEM((2, _BLK, _D), jnp.float32),
            pltpu.SemaphoreType.DMA,
            pltpu.SemaphoreType.DMA,
            pltpu.SemaphoreType.DMA,
            pltpu.SemaphoreType.DMA,
        ],
    )
    def sc_kernel(x_hbm, src_hbm, dst_hbm, rel_hbm, out_hbm,
                  sidx_v, didx_v, out_ring, head_v, tail_v, rel_v,
                  sem0, sem1, sem_o0, sem_o1):
        wid = lax.axis_index("s") * _NC + lax.axis_index("c")
        base = wid * _E_PER_W

        pltpu.sync_copy(src_hbm.at[pl.ds(base, _E_PER_W)], sidx_v)
        pltpu.sync_copy(dst_hbm.at[pl.ds(base, _E_PER_W)], didx_v)

        sems = (sem0, sem1)
        out_sems = (sem_o0, sem_o1)
        lane = lax.iota(jnp.int32, _L)
        rot_idx = [(lane + sh) & (_L - 1) for sh in (8, 4, 2, 1)]
        gd = lax.GatherDimensionNumbers(
            offset_dims=(), collapsed_slice_dims=(0,), start_index_map=(0,))

        def _rot(v, idx):
            return lax.gather(v, idx[:, None], gd, (1,),
                              mode=lax.GatherScatterMode.PROMISE_IN_BOUNDS)

        def start(b, slot):
            off = b * _BLK
            pltpu.async_copy(
                x_hbm.at[sidx_v.at[pl.ds(off, _BLK)]],
                head_v.at[slot], sems[slot])
            pltpu.async_copy(
                x_hbm.at[didx_v.at[pl.ds(off, _BLK)]],
                tail_v.at[slot], sems[slot])
            pltpu.async_copy(
                rel_hbm.at[pl.ds(base + off, _BLK)],
                rel_v.at[slot], sems[slot])

        def wait(b, slot):
            off = b * _BLK
            pltpu.make_async_copy(
                x_hbm.at[sidx_v.at[pl.ds(off, _BLK)]],
                head_v.at[slot], sems[slot]).wait()
            pltpu.make_async_copy(
                x_hbm.at[didx_v.at[pl.ds(off, _BLK)]],
                tail_v.at[slot], sems[slot]).wait()
            pltpu.make_async_copy(
                rel_hbm.at[pl.ds(base + off, _BLK)],
                rel_v.at[slot], sems[slot]).wait()

        def compute(b, slot):
            off = b * _BLK
            hb, tb, rb = head_v.at[slot], tail_v.at[slot], rel_v.at[slot]
            ob = out_ring.at[slot]

            @pl.when(b >= 2)
            def _():
                pltpu.make_async_copy(
                    ob, out_hbm.at[pl.ds(base + (b - 2) * _BLK, _BLK)],
                    out_sems[slot]).wait()

            hi_mask = jnp.full((_L,), -65536, jnp.int32)

            def group_body(g, carry2):
                row0 = g * _L
                vec = jnp.zeros((_L,), jnp.float32)
                for e in range(_L):
                    r = row0 + e
                    acc = jnp.zeros((_L,), jnp.float32)
                    for c in range(_D // (2 * _L)):
                        hw = hb[r, pl.ds(c * _L, _L)]
                        tw = tb[r, pl.ds(c * _L, _L)]
                        d = (plsc.bitcast(hw, jnp.bfloat16)
                             - plsc.bitcast(tw, jnp.bfloat16))
                        dw = plsc.bitcast(d, jnp.int32)
                        d0 = lax.bitcast_convert_type(dw << 16, jnp.float32)
                        d1 = lax.bitcast_convert_type(dw & hi_mask, jnp.float32)
                        r0 = rb[r, pl.ds((2 * c) * _L, _L)]
                        r1 = rb[r, pl.ds((2 * c + 1) * _L, _L)]
                        acc = acc + jnp.abs(d0 + r0)
                        acc = acc + jnp.abs(d1 + r1)
                    for idx in rot_idx:
                        acc = acc + _rot(acc, idx)
                    vec = jnp.where(lane == e, _GAMMA - acc, vec)
                ob[pl.ds(row0, _L)] = vec
                return carry2

            lax.fori_loop(0, _BLK // _L, group_body, 0)
            pltpu.async_copy(
                ob, out_hbm.at[pl.ds(base + off, _BLK)], out_sems[slot])

        start(0, 0)

        def pair_body(g, carry):
            b0 = 2 * g
            start(b0 + 1, 1)
            wait(b0, 0)
            compute(b0, 0)
            start(b0 + 2, 0)
            wait(b0 + 1, 1)
            compute(b0 + 1, 1)
            return carry

        lax.fori_loop(0, (_NBLK - 1) // 2, pair_body, 0)
        wait(_NBLK - 1, 0)
        compute(_NBLK - 1, 0)

        pltpu.make_async_copy(
            out_ring.at[1],
            out_hbm.at[pl.ds(base + (_NBLK - 2) * _BLK, _BLK)],
            out_sems[1]).wait()
        pltpu.make_async_copy(
            out_ring.at[0],
            out_hbm.at[pl.ds(base + (_NBLK - 1) * _BLK, _BLK)],
            out_sems[0]).wait()

    return sc_kernel


_sc_kernel = _make_sc_kernel()


@jax.jit
def kernel(x, edge_index, rel_emb):
    src = edge_index[0].astype(jnp.int32)
    dst = edge_index[1].astype(jnp.int32)
    n = x.shape[0]
    x_perm = x.reshape(n, _D // 32, 2, _L).swapaxes(2, 3).reshape(n, _D)
    x_bf = jnp.pad(x_perm, ((0, _N_PAD - n), (0, 0))).astype(jnp.bfloat16)
    x_i32 = jax.lax.bitcast_convert_type(
        x_bf.reshape(_N_PAD, _D // 2, 2), jnp.int32)
    return _sc_kernel(x_i32, src, dst, rel_emb)
